# baseline (speedup 1.0000x reference)
import functools

import jax
import jax.numpy as jnp
from jax import lax
from jax.experimental import pallas as pl
from jax.experimental.pallas import tpu as pltpu
from jax.experimental.pallas import tpu_sc as plsc

_NW = 32
_LANES = 16
_N_AST = 100000


def _build_src_map(key_i32, val_i32, n_ast, n_cfg):
    m = key_i32.shape[0]
    assert m % _LANES == 0
    per = -(-n_ast // (_NW * _LANES)) * _LANES
    s_pad = per * _NW
    mesh = plsc.VectorSubcoreMesh(core_axis_name="c", subcore_axis_name="s")

    @functools.partial(
        pl.kernel,
        out_type=jax.ShapeDtypeStruct((s_pad,), jnp.int32),
        mesh=mesh,
        scratch_types=[
            pltpu.VMEM((m,), jnp.int32),
            pltpu.VMEM((m,), jnp.int32),
            pltpu.VMEM((per,), jnp.int32),
        ],
        compiler_params=pltpu.CompilerParams(needs_layout_passes=False),
    )
    def build(key_hbm, val_hbm, src_hbm, keys_v, vals_v, s_v):
        wid = lax.axis_index("s") * 2 + lax.axis_index("c")
        lo = wid * per

        def init_body(i, _):
            s_v[pl.ds(i * _LANES, _LANES)] = jnp.full((_LANES,), n_cfg, jnp.int32)
            return _

        lax.fori_loop(0, per // _LANES, init_body, None)

        pltpu.sync_copy(key_hbm, keys_v)
        pltpu.sync_copy(val_hbm, vals_v)

        def scan_body(i, _):
            k16 = keys_v[pl.ds(i * _LANES, _LANES)]
            inr = (k16 >= lo) & (k16 < lo + per)
            loc = jnp.where(inr, k16 - lo, 0)
            v16 = vals_v[pl.ds(i * _LANES, _LANES)]
            plsc.store_scatter(s_v, [loc], v16, mask=inr)
            return _

        lax.fori_loop(0, m // _LANES, scan_body, None)

        pltpu.sync_copy(s_v, src_hbm.at[pl.ds(lo, per)])

    return build(key_i32, val_i32)


def _gather_updates(src_map, path_i32, cfg_ext):
    nt = path_i32.shape[0]
    s_pad = src_map.shape[0]
    d = cfg_ext.shape[1]
    per = nt // _NW
    assert per * _NW == nt and per % 8 == 0
    ch = 64
    nfull, rem = per // ch, per % ch
    assert nfull % 2 == 0 and rem % 8 == 0
    nvec = per // _LANES
    vtail = per - nvec * _LANES
    mesh = plsc.VectorSubcoreMesh(core_axis_name="c", subcore_axis_name="s")

    @functools.partial(
        pl.kernel,
        out_type=jax.ShapeDtypeStruct((nt, d), jnp.float32),
        mesh=mesh,
        scratch_types=[
            pltpu.VMEM((s_pad,), jnp.int32),
            pltpu.VMEM((per,), jnp.int32),
            pltpu.VMEM((per,), jnp.int32),
            pltpu.VMEM((2, ch, d), jnp.float32),
            pltpu.SemaphoreType.DMA,
            pltpu.SemaphoreType.DMA,
        ],
        compiler_params=pltpu.CompilerParams(needs_layout_passes=False),
    )
    def gather(src_hbm, path_hbm, cfg_hbm, upd_hbm, src_v, pidx_v, g_v, rows_v,
               sem0, sem1):
        wid = lax.axis_index("s") * 2 + lax.axis_index("c")
        base = wid * per
        pltpu.sync_copy(src_hbm, src_v)
        pltpu.sync_copy(path_hbm.at[pl.ds(base, per)], pidx_v)

        def lookup_body(i, _):
            p16 = pidx_v[pl.ds(i * _LANES, _LANES)]
            g_v[pl.ds(i * _LANES, _LANES)] = plsc.load_gather(src_v, [p16])
            return _

        lax.fori_loop(0, nvec, lookup_body, None)
        if vtail:
            p16 = pidx_v[pl.ds(per - _LANES, _LANES)]
            g_v[pl.ds(per - _LANES, _LANES)] = plsc.load_gather(src_v, [p16])

        sems = (sem0, sem1)

        def start(c, b):
            off = pl.multiple_of(c * ch, ch)
            pltpu.async_copy(
                cfg_hbm.at[g_v.at[pl.ds(off, ch)]], rows_v.at[b], sems[b])

        def finish(c, b):
            off = pl.multiple_of(c * ch, ch)
            pltpu.make_async_copy(
                cfg_hbm.at[g_v.at[pl.ds(off, ch)]], rows_v.at[b], sems[b]
            ).wait()
            pltpu.sync_copy(rows_v.at[b], upd_hbm.at[pl.ds(base + off, ch)])

        start(0, 0)

        def ring_body(it, _):
            c0 = it * 2

            @pl.when(c0 + 1 < nfull)
            def _():
                start(c0 + 1, 1)

            finish(c0, 0)

            @pl.when(c0 + 2 < nfull)
            def _():
                start(c0 + 2, 0)

            @pl.when(c0 + 1 < nfull)
            def _():
                finish(c0 + 1, 1)

            return _

        lax.fori_loop(0, nfull // 2, ring_body, None)

        if rem:
            pltpu.async_copy(
                cfg_hbm.at[g_v.at[pl.ds(nfull * ch, rem)]],
                rows_v.at[0].at[pl.ds(0, rem)], sem0).wait()
            pltpu.sync_copy(rows_v.at[0].at[pl.ds(0, rem)],
                            upd_hbm.at[pl.ds(base + nfull * ch, rem)])

    return gather(src_map, path_i32, cfg_ext)


def _cat_project(prev2d, upd2d, w1, w2, b2d):
    nt, d = prev2d.shape
    blk = 2000
    assert nt % blk == 0

    def body(prev_ref, upd_ref, w1_ref, w2_ref, b_ref, out_ref):
        acc = jnp.dot(prev_ref[...], w1_ref[...], preferred_element_type=jnp.float32)
        acc += jnp.dot(upd_ref[...], w2_ref[...], preferred_element_type=jnp.float32)
        out_ref[...] = jnp.maximum(acc + b_ref[...], 0.0)

    return pl.pallas_call(
        body,
        grid=(nt // blk,),
        in_specs=[
            pl.BlockSpec((blk, d), lambda i: (i, 0)),
            pl.BlockSpec((blk, d), lambda i: (i, 0)),
            pl.BlockSpec((d, d), lambda i: (0, 0)),
            pl.BlockSpec((d, d), lambda i: (0, 0)),
            pl.BlockSpec((1, d), lambda i: (0, 0)),
        ],
        out_specs=pl.BlockSpec((blk, d), lambda i: (i, 0)),
        out_shape=jax.ShapeDtypeStruct((nt, d), jnp.float32),
    )(prev2d, upd2d, w1, w2, b2d)


def kernel(nr_ast_nodes, prev_nodes_occurrences, new_cfg_nodes_encodings,
           mapping_value_indices, mapping_key_indices, path_node_indices, W, b):
    p, l, d = prev_nodes_occurrences.shape
    n_cfg = new_cfg_nodes_encodings.shape[0]
    nt = p * l

    key_i32 = jnp.minimum(mapping_key_indices, nr_ast_nodes - 1).astype(jnp.int32)
    val_i32 = mapping_value_indices.astype(jnp.int32)
    path_flat = path_node_indices.reshape(nt).astype(jnp.int32)
    cfg_ext = jnp.concatenate(
        [new_cfg_nodes_encodings,
         jnp.zeros((8, d), new_cfg_nodes_encodings.dtype)], axis=0)

    src_map = _build_src_map(key_i32, val_i32, _N_AST, n_cfg)
    upd2d = _gather_updates(src_map, path_flat, cfg_ext)

    prev2d = prev_nodes_occurrences.reshape(nt, d)
    out2d = _cat_project(prev2d, upd2d, W[:d], W[d:], b.reshape(1, d))
    return out2d.reshape(p, l, d)


# speedup vs baseline: 1.4468x; 1.4468x over previous
import functools

import jax
import jax.numpy as jnp
from jax import lax
from jax.experimental import pallas as pl
from jax.experimental.pallas import tpu as pltpu
from jax.experimental.pallas import tpu_sc as plsc

_NW = 32
_LANES = 16
_N_AST = 100000


def _build_src_map(key_i32, val_i32, n_ast, n_cfg):
    m = key_i32.shape[0]
    assert m % _LANES == 0
    per = -(-n_ast // (_NW * _LANES)) * _LANES
    s_pad = per * _NW
    mesh = plsc.VectorSubcoreMesh(core_axis_name="c", subcore_axis_name="s")

    @functools.partial(
        pl.kernel,
        out_type=jax.ShapeDtypeStruct((s_pad,), jnp.int32),
        mesh=mesh,
        scratch_types=[
            pltpu.VMEM((m,), jnp.int32),
            pltpu.VMEM((m,), jnp.int32),
            pltpu.VMEM((per,), jnp.int32),
        ],
        compiler_params=pltpu.CompilerParams(needs_layout_passes=False),
    )
    def build(key_hbm, val_hbm, src_hbm, keys_v, vals_v, s_v):
        wid = lax.axis_index("s") * 2 + lax.axis_index("c")
        lo = wid * per

        def init_body(i, _):
            s_v[pl.ds(i * _LANES, _LANES)] = jnp.full((_LANES,), n_cfg, jnp.int32)
            return _

        lax.fori_loop(0, per // _LANES, init_body, None)

        pltpu.sync_copy(key_hbm, keys_v)
        pltpu.sync_copy(val_hbm, vals_v)

        def scan_body(i, _):
            k16 = keys_v[pl.ds(i * _LANES, _LANES)]
            inr = (k16 >= lo) & (k16 < lo + per)
            loc = jnp.where(inr, k16 - lo, 0)
            v16 = vals_v[pl.ds(i * _LANES, _LANES)]
            plsc.store_scatter(s_v, [loc], v16, mask=inr)
            return _

        lax.fori_loop(0, m // _LANES, scan_body, None)

        pltpu.sync_copy(s_v, src_hbm.at[pl.ds(lo, per)])

    return build(key_i32, val_i32)


def _gather_updates(src_map, path_i32, cfg_ext):
    nt = path_i32.shape[0]
    s_pad = src_map.shape[0]
    d = cfg_ext.shape[1]
    per = nt // _NW
    assert per * _NW == nt and per % 8 == 0
    ch = 64
    nfull, rem = per // ch, per % ch
    assert nfull % 2 == 0 and rem % 8 == 0
    nvec = per // _LANES
    vtail = per - nvec * _LANES
    mesh = plsc.VectorSubcoreMesh(core_axis_name="c", subcore_axis_name="s")

    @functools.partial(
        pl.kernel,
        out_type=jax.ShapeDtypeStruct((nt, d), jnp.float32),
        mesh=mesh,
        scratch_types=[
            pltpu.VMEM((s_pad,), jnp.int32),
            pltpu.VMEM((per,), jnp.int32),
            pltpu.VMEM((per,), jnp.int32),
            pltpu.VMEM((2, ch, d), jnp.float32),
            pltpu.SemaphoreType.DMA,
            pltpu.SemaphoreType.DMA,
        ],
        compiler_params=pltpu.CompilerParams(
            needs_layout_passes=False, use_tc_tiling_on_sc=False),
    )
    def gather(src_hbm, path_hbm, cfg_hbm, upd_hbm, src_v, pidx_v, g_v, rows_v,
               sem0, sem1):
        wid = lax.axis_index("s") * 2 + lax.axis_index("c")
        base = wid * per
        pltpu.sync_copy(src_hbm, src_v)
        pltpu.sync_copy(path_hbm.at[pl.ds(base, per)], pidx_v)

        def lookup_body(i, _):
            p16 = pidx_v[pl.ds(i * _LANES, _LANES)]
            g_v[pl.ds(i * _LANES, _LANES)] = plsc.load_gather(src_v, [p16])
            return _

        lax.fori_loop(0, nvec, lookup_body, None)
        if vtail:
            p16 = pidx_v[pl.ds(per - _LANES, _LANES)]
            g_v[pl.ds(per - _LANES, _LANES)] = plsc.load_gather(src_v, [p16])

        sems = (sem0, sem1)

        def start(c, b):
            off = pl.multiple_of(c * ch, ch)
            pltpu.async_copy(
                cfg_hbm.at[g_v.at[pl.ds(off, ch)]], rows_v.at[b], sems[b])

        def finish(c, b):
            off = pl.multiple_of(c * ch, ch)
            pltpu.make_async_copy(
                cfg_hbm.at[g_v.at[pl.ds(off, ch)]], rows_v.at[b], sems[b]
            ).wait()
            pltpu.sync_copy(rows_v.at[b], upd_hbm.at[pl.ds(base + off, ch)])

        start(0, 0)

        def ring_body(it, _):
            c0 = it * 2

            @pl.when(c0 + 1 < nfull)
            def _():
                start(c0 + 1, 1)

            finish(c0, 0)

            @pl.when(c0 + 2 < nfull)
            def _():
                start(c0 + 2, 0)

            @pl.when(c0 + 1 < nfull)
            def _():
                finish(c0 + 1, 1)

            return _

        lax.fori_loop(0, nfull // 2, ring_body, None)

        if rem:
            pltpu.async_copy(
                cfg_hbm.at[g_v.at[pl.ds(nfull * ch, rem)]],
                rows_v.at[0].at[pl.ds(0, rem)], sem0).wait()
            pltpu.sync_copy(rows_v.at[0].at[pl.ds(0, rem)],
                            upd_hbm.at[pl.ds(base + nfull * ch, rem)])

    return gather(src_map, path_i32, cfg_ext)


def _cat_project(prev2d, upd2d, w1, w2, b2d):
    nt, d = prev2d.shape
    blk = 2000
    assert nt % blk == 0

    def body(prev_ref, upd_ref, w1_ref, w2_ref, b_ref, out_ref):
        acc = jnp.dot(prev_ref[...], w1_ref[...], preferred_element_type=jnp.float32)
        acc += jnp.dot(upd_ref[...], w2_ref[...], preferred_element_type=jnp.float32)
        out_ref[...] = jnp.maximum(acc + b_ref[...], 0.0)

    return pl.pallas_call(
        body,
        grid=(nt // blk,),
        in_specs=[
            pl.BlockSpec((blk, d), lambda i: (i, 0)),
            pl.BlockSpec((blk, d), lambda i: (i, 0)),
            pl.BlockSpec((d, d), lambda i: (0, 0)),
            pl.BlockSpec((d, d), lambda i: (0, 0)),
            pl.BlockSpec((1, d), lambda i: (0, 0)),
        ],
        out_specs=pl.BlockSpec((blk, d), lambda i: (i, 0)),
        out_shape=jax.ShapeDtypeStruct((nt, d), jnp.float32),
    )(prev2d, upd2d, w1, w2, b2d)


def kernel(nr_ast_nodes, prev_nodes_occurrences, new_cfg_nodes_encodings,
           mapping_value_indices, mapping_key_indices, path_node_indices, W, b):
    p, l, d = prev_nodes_occurrences.shape
    n_cfg = new_cfg_nodes_encodings.shape[0]
    nt = p * l

    key_i32 = jnp.minimum(mapping_key_indices, nr_ast_nodes - 1).astype(jnp.int32)
    val_i32 = mapping_value_indices.astype(jnp.int32)
    path_flat = path_node_indices.reshape(nt).astype(jnp.int32)
    cfg_bf = jnp.concatenate(
        [new_cfg_nodes_encodings.astype(jnp.bfloat16),
         jnp.zeros((8, d), jnp.bfloat16)], axis=0)
    cfg_words = jax.lax.bitcast_convert_type(
        cfg_bf.reshape(n_cfg + 8, d // 2, 2), jnp.float32)

    src_map = _build_src_map(key_i32, val_i32, _N_AST, n_cfg)
    upd_words = _gather_updates(src_map, path_flat, cfg_words)
    upd_bf = jax.lax.bitcast_convert_type(upd_words, jnp.bfloat16).reshape(nt, d)

    prev2d = prev_nodes_occurrences.reshape(nt, d)
    out2d = _cat_project(prev2d, upd_bf, W[:d],
                         W[d:].astype(jnp.bfloat16), b.reshape(1, d))
    return out2d.reshape(p, l, d)
